# baseline (speedup 1.0000x reference)
import functools

import jax
import jax.numpy as jnp
from jax import lax
from jax.experimental import pallas as pl
from jax.experimental.pallas import tpu as pltpu
from jax.experimental.pallas import tpu_sc as plsc

_B = 1024
_FDIM = 128
_P = 100000
_Q = 5000
_TEMPERATURE = -0.01
_OIM_SCALAR = 30.0
_LOG2E = 1.4426950408889634
_LN2 = 0.6931471805599453

_NBLK = 8192
_NL = (_P + _NBLK - 1) // _NBLK
_NC = (_Q + _NBLK - 1) // _NBLK
_NT = _NL + _NC
_NEG = -1e30

_RELW = 128


def _make_sc_gather():
    info = plsc.get_sparse_core_info()
    nc, ns = info.num_cores, info.num_subcores
    nw = nc * ns
    b_per_w = _B // nw

    @functools.partial(
        pl.kernel,
        mesh=plsc.VectorSubcoreMesh(core_axis_name="c", subcore_axis_name="s"),
        out_type=[
            jax.ShapeDtypeStruct((_B, _FDIM), jnp.float32),
            jax.ShapeDtypeStruct((_B, _RELW), jnp.float32),
        ],
        scratch_types=[
            pltpu.VMEM((b_per_w,), jnp.int32),
            pltpu.VMEM((b_per_w,), jnp.int32),
            pltpu.VMEM((b_per_w, _FDIM), jnp.float32),
            pltpu.VMEM((b_per_w, _RELW), jnp.float32),
            pltpu.SemaphoreType.DMA,
            pltpu.SemaphoreType.DMA,
        ],
    )
    def sc_gather(lut_hbm, rel2d_hbm, sl_hbm, ri_hbm, g_out, r_out,
                  idx_v, ridx_v, rows_v, rel_v, sem0, sem1):
        wid = lax.axis_index("s") * nc + lax.axis_index("c")
        base = wid * b_per_w
        pltpu.sync_copy(sl_hbm.at[pl.ds(base, b_per_w)], idx_v)
        pltpu.sync_copy(ri_hbm.at[pl.ds(base, b_per_w)], ridx_v)
        cp0 = pltpu.async_copy(lut_hbm.at[idx_v], rows_v, sem0)
        cp1 = pltpu.async_copy(rel2d_hbm.at[ridx_v], rel_v, sem1)
        cp0.wait()
        cp1.wait()
        pltpu.sync_copy(rows_v, g_out.at[pl.ds(base, b_per_w)])
        pltpu.sync_copy(rel_v, r_out.at[pl.ds(base, b_per_w)])

    return sc_gather


def _oim_kernel(x_ref, lut_ref, cq_ref, rel_ref, lab_ref, mod_ref,
                g_ref, r16_ref, out_ref, m_ref, s_ref):
    n = pl.program_id(0)

    @pl.when(n == 0)
    def _init():
        m_ref[:, :] = jnp.full((_B, 1), _NEG, jnp.float32)
        s_ref[:, :] = jnp.zeros((_B, 1), jnp.float32)

    def process(blk_ref, n_valid):
        scale = jnp.exp(_TEMPERATURE * rel_ref[0, :, :]) * (_OIM_SCALAR * _LOG2E)
        blk = (blk_ref[:, :] * scale).astype(jnp.bfloat16)
        half = _B // 4
        for lo in (0, half, 2 * half, 3 * half):
            logits = lax.dot_general(
                x_ref[lo:lo + half, :], blk,
                dimension_numbers=(((1,), (1,)), ((), ())),
                preferred_element_type=jnp.float32,
            ).astype(jnp.bfloat16)
            if n_valid is not None:
                local_col = lax.broadcasted_iota(jnp.int32, (1, _NBLK), 1)
                logits = jnp.where(local_col < n_valid, logits, _NEG)
            t = logits
            while t.shape[1] > 128:
                w = t.shape[1] // 2
                t = jnp.maximum(t[:, :w], t[:, w:])
            mx = jnp.max(t.astype(jnp.float32), axis=1, keepdims=True)
            m_old = m_ref[lo:lo + half, :]
            m_new = jnp.maximum(m_old, mx)
            mb = m_new.astype(jnp.bfloat16)
            w = _NBLK // 2
            e = (jnp.exp2(logits[:, :w] - mb) + jnp.exp2(logits[:, w:] - mb))
            while e.shape[1] > 128:
                w = e.shape[1] // 2
                e = e[:, :w] + e[:, w:]
            s_part = jnp.sum(e.astype(jnp.float32), axis=1, keepdims=True)
            s_ref[lo:lo + half, :] = (s_ref[lo:lo + half, :]
                                      * jnp.exp2(m_old - m_new) + s_part)
            m_ref[lo:lo + half, :] = m_new

    @pl.when(n < _NL - 1)
    def _lut_step():
        process(lut_ref, None)

    @pl.when(n == _NL - 1)
    def _lut_last():
        process(lut_ref, _P - (_NL - 1) * _NBLK)

    @pl.when((n >= _NL) & (n < _NT - 1))
    def _cq_step():
        process(cq_ref, None)

    @pl.when(n == _NT - 1)
    def _cq_last():
        process(cq_ref, _Q - (_NC - 1) * _NBLK)

    @pl.when(n == _NT - 1)
    def _finish():
        lab = lab_ref[:, :] - 1
        u = jnp.sum(x_ref[:, :].astype(jnp.float32) * g_ref[:, :],
                    axis=1, keepdims=True)
        lane = lax.broadcasted_iota(jnp.int32, (1, _RELW), 1)
        relv = jnp.sum(jnp.where(lane == mod_ref[:, :], r16_ref[:, :], 0.0),
                       axis=1, keepdims=True)
        t2 = u * (jnp.exp(_TEMPERATURE * relv) * (_OIM_SCALAR * _LOG2E))
        lse2 = m_ref[:, :] + jnp.log2(s_ref[:, :])
        nll = (lse2 - t2) * _LN2
        mask = ((lab >= 0) & (lab != 5554)).astype(jnp.float32)
        num = jnp.sum(nll * mask, keepdims=True)
        den = jnp.sum(mask, keepdims=True)
        out_ref[:, :] = num / jnp.maximum(den, 1.0)


def kernel(inputs, roi_label, roi_ious, lut, cq, reliability):
    del roi_ious
    x = inputs.reshape(_B, _FDIM).astype(jnp.bfloat16)
    lab2d = roi_label.reshape(_B, 1).astype(jnp.int32)
    lab_flat = roi_label.reshape(_B).astype(jnp.int32)
    safe_lab = jnp.maximum(lab_flat - 1, 0)
    rel_row = safe_lab // _RELW
    rel_mod = (safe_lab % _RELW).reshape(_B, 1)
    rel2d = jnp.pad(reliability[:_P], (0, -_P % _RELW)).reshape(-1, _RELW)

    g_rows, r16 = _make_sc_gather()(lut, rel2d, safe_lab, rel_row)

    rel_l = jnp.pad(reliability[:_P], (0, _NL * _NBLK - _P)).reshape(_NL, _NBLK, 1)
    rel_c = jnp.pad(reliability[_P:], (0, _NC * _NBLK - _Q)).reshape(_NC, _NBLK, 1)
    rel_pad = jnp.concatenate([rel_l, rel_c], axis=0)

    out = pl.pallas_call(
        _oim_kernel,
        grid=(_NT,),
        in_specs=[
            pl.BlockSpec((_B, _FDIM), lambda n: (0, 0)),
            pl.BlockSpec((_NBLK, _FDIM), lambda n: (jnp.minimum(n, _NL - 1), 0)),
            pl.BlockSpec((_NBLK, _FDIM), lambda n: (jnp.maximum(n - _NL, 0), 0)),
            pl.BlockSpec((1, _NBLK, 1), lambda n: (n, 0, 0)),
            pl.BlockSpec((_B, 1), lambda n: (0, 0)),
            pl.BlockSpec((_B, 1), lambda n: (0, 0)),
            pl.BlockSpec((_B, _FDIM), lambda n: (0, 0)),
            pl.BlockSpec((_B, _RELW), lambda n: (0, 0)),
        ],
        out_specs=pl.BlockSpec((1, 1), lambda n: (0, 0)),
        out_shape=jax.ShapeDtypeStruct((1, 1), jnp.float32),
        scratch_shapes=[
            pltpu.VMEM((_B, 1), jnp.float32),
            pltpu.VMEM((_B, 1), jnp.float32),
        ],
        compiler_params=pltpu.CompilerParams(
            dimension_semantics=("arbitrary",),
        ),
    )(x, lut, cq, rel_pad, lab2d, rel_mod, g_rows, r16)
    return out[0, 0]


# speedup vs baseline: 2.4014x; 2.4014x over previous
import functools

import jax
import jax.numpy as jnp
from jax import lax
from jax.experimental import pallas as pl
from jax.experimental.pallas import tpu as pltpu
from jax.experimental.pallas import tpu_sc as plsc

_B = 1024
_FDIM = 128
_P = 100000
_Q = 5000
_TEMPERATURE = -0.01
_OIM_SCALAR = 30.0
_LOG2E = 1.4426950408889634
_LN2 = 0.6931471805599453

_NBLK = 4096
_NL = (_P + _NBLK - 1) // _NBLK
_NC = (_Q + _NBLK - 1) // _NBLK
_NT = _NL + _NC
_NEG = -1e30

_RELW = 128


def _make_sc_gather():
    info = plsc.get_sparse_core_info()
    nc, ns = info.num_cores, info.num_subcores
    nw = nc * ns
    b_per_w = _B // nw

    @functools.partial(
        pl.kernel,
        mesh=plsc.VectorSubcoreMesh(core_axis_name="c", subcore_axis_name="s"),
        out_type=[
            jax.ShapeDtypeStruct((_B, _FDIM), jnp.float32),
            jax.ShapeDtypeStruct((_B, _RELW), jnp.float32),
        ],
        scratch_types=[
            pltpu.VMEM((b_per_w,), jnp.int32),
            pltpu.VMEM((b_per_w,), jnp.int32),
            pltpu.VMEM((b_per_w, _FDIM), jnp.float32),
            pltpu.VMEM((b_per_w, _RELW), jnp.float32),
            pltpu.SemaphoreType.DMA,
            pltpu.SemaphoreType.DMA,
        ],
    )
    def sc_gather(lut_hbm, rel2d_hbm, sl_hbm, ri_hbm, g_out, r_out,
                  idx_v, ridx_v, rows_v, rel_v, sem0, sem1):
        wid = lax.axis_index("s") * nc + lax.axis_index("c")
        base = wid * b_per_w
        pltpu.sync_copy(sl_hbm.at[pl.ds(base, b_per_w)], idx_v)
        pltpu.sync_copy(ri_hbm.at[pl.ds(base, b_per_w)], ridx_v)
        cp0 = pltpu.async_copy(lut_hbm.at[idx_v], rows_v, sem0)
        cp1 = pltpu.async_copy(rel2d_hbm.at[ridx_v], rel_v, sem1)
        cp0.wait()
        cp1.wait()
        pltpu.sync_copy(rows_v, g_out.at[pl.ds(base, b_per_w)])
        pltpu.sync_copy(rel_v, r_out.at[pl.ds(base, b_per_w)])

    return sc_gather


def _oim_kernel(x_ref, lut_ref, cq_ref, rel_ref, lab_ref, mod_ref,
                g_ref, r16_ref, out_ref, m_ref, s_ref):
    n = pl.program_id(0)

    @pl.when(n == 0)
    def _init():
        m_ref[:, :] = jnp.full((_B, 1), _NEG, jnp.float32)
        s_ref[:, :] = jnp.zeros((_B, 1), jnp.float32)

    def process(blk_ref, n_valid):
        scale = jnp.exp(_TEMPERATURE * rel_ref[0, :, :]) * (_OIM_SCALAR * _LOG2E)
        blk = (blk_ref[:, :] * scale).astype(jnp.bfloat16)
        half = _B // 4
        for lo in (0, half, 2 * half, 3 * half):
            logits = lax.dot_general(
                x_ref[lo:lo + half, :], blk,
                dimension_numbers=(((1,), (1,)), ((), ())),
                preferred_element_type=jnp.float32,
            ).astype(jnp.bfloat16)
            if n_valid is not None:
                local_col = lax.broadcasted_iota(jnp.int32, (1, _NBLK), 1)
                logits = jnp.where(local_col < n_valid, logits, _NEG)
            t = logits
            while t.shape[1] > 128:
                w = t.shape[1] // 2
                t = jnp.maximum(t[:, :w], t[:, w:])
            mx = jnp.max(t.astype(jnp.float32), axis=1, keepdims=True)
            m_old = m_ref[lo:lo + half, :]
            m_new = jnp.maximum(m_old, mx)
            mb = m_new.astype(jnp.bfloat16)
            w = _NBLK // 2
            e = (jnp.exp2(logits[:, :w] - mb) + jnp.exp2(logits[:, w:] - mb))
            while e.shape[1] > 128:
                w = e.shape[1] // 2
                e = e[:, :w] + e[:, w:]
            s_part = jnp.sum(e.astype(jnp.float32), axis=1, keepdims=True)
            s_ref[lo:lo + half, :] = (s_ref[lo:lo + half, :]
                                      * jnp.exp2(m_old - m_new) + s_part)
            m_ref[lo:lo + half, :] = m_new

    @pl.when(n < _NL - 1)
    def _lut_step():
        process(lut_ref, None)

    @pl.when(n == _NL - 1)
    def _lut_last():
        process(lut_ref, _P - (_NL - 1) * _NBLK)

    @pl.when((n >= _NL) & (n < _NT - 1))
    def _cq_step():
        process(cq_ref, None)

    @pl.when(n == _NT - 1)
    def _cq_last():
        process(cq_ref, _Q - (_NC - 1) * _NBLK)

    @pl.when(n == _NT - 1)
    def _finish():
        lab = lab_ref[:, :] - 1
        u = jnp.sum(x_ref[:, :].astype(jnp.float32) * g_ref[:, :],
                    axis=1, keepdims=True)
        lane = lax.broadcasted_iota(jnp.int32, (1, _RELW), 1)
        relv = jnp.sum(jnp.where(lane == mod_ref[:, :], r16_ref[:, :], 0.0),
                       axis=1, keepdims=True)
        t2 = u * (jnp.exp(_TEMPERATURE * relv) * (_OIM_SCALAR * _LOG2E))
        lse2 = m_ref[:, :] + jnp.log2(s_ref[:, :])
        nll = (lse2 - t2) * _LN2
        mask = ((lab >= 0) & (lab != 5554)).astype(jnp.float32)
        num = jnp.sum(nll * mask, keepdims=True)
        den = jnp.sum(mask, keepdims=True)
        out_ref[:, :] = num / jnp.maximum(den, 1.0)


def kernel(inputs, roi_label, roi_ious, lut, cq, reliability):
    del roi_ious
    x = inputs.reshape(_B, _FDIM).astype(jnp.bfloat16)
    lab2d = roi_label.reshape(_B, 1).astype(jnp.int32)
    lab_flat = roi_label.reshape(_B).astype(jnp.int32)
    safe_lab = jnp.maximum(lab_flat - 1, 0)
    rel_row = safe_lab // _RELW
    rel_mod = (safe_lab % _RELW).reshape(_B, 1)
    rel2d = jnp.pad(reliability[:_P], (0, -_P % _RELW)).reshape(-1, _RELW)

    g_rows, r16 = _make_sc_gather()(lut, rel2d, safe_lab, rel_row)

    rel_l = jnp.pad(reliability[:_P], (0, _NL * _NBLK - _P)).reshape(_NL, _NBLK, 1)
    rel_c = jnp.pad(reliability[_P:], (0, _NC * _NBLK - _Q)).reshape(_NC, _NBLK, 1)
    rel_pad = jnp.concatenate([rel_l, rel_c], axis=0)

    out = pl.pallas_call(
        _oim_kernel,
        grid=(_NT,),
        in_specs=[
            pl.BlockSpec((_B, _FDIM), lambda n: (0, 0)),
            pl.BlockSpec((_NBLK, _FDIM), lambda n: (jnp.minimum(n, _NL - 1), 0)),
            pl.BlockSpec((_NBLK, _FDIM), lambda n: (jnp.maximum(n - _NL, 0), 0)),
            pl.BlockSpec((1, _NBLK, 1), lambda n: (n, 0, 0)),
            pl.BlockSpec((_B, 1), lambda n: (0, 0)),
            pl.BlockSpec((_B, 1), lambda n: (0, 0)),
            pl.BlockSpec((_B, _FDIM), lambda n: (0, 0)),
            pl.BlockSpec((_B, _RELW), lambda n: (0, 0)),
        ],
        out_specs=pl.BlockSpec((1, 1), lambda n: (0, 0)),
        out_shape=jax.ShapeDtypeStruct((1, 1), jnp.float32),
        scratch_shapes=[
            pltpu.VMEM((_B, 1), jnp.float32),
            pltpu.VMEM((_B, 1), jnp.float32),
        ],
        compiler_params=pltpu.CompilerParams(
            dimension_semantics=("arbitrary",),
        ),
    )(x, lut, cq, rel_pad, lab2d, rel_mod, g_rows, r16)
    return out[0, 0]
